# Initial kernel scaffold; baseline (speedup 1.0000x reference)
import jax
import jax.numpy as jnp
from jax.experimental import pallas as pl


def kernel(inputs):
    raise NotImplementedError("write your pallas kernel here")


# speedup vs baseline: 1.3043x; 1.3043x over previous
import functools

import numpy as np
import jax
import jax.numpy as jnp
from jax import lax
from jax.experimental import pallas as pl
from jax.experimental.pallas import tpu as pltpu
from jax.experimental.pallas import tpu_sc as plsc

B, H, W, C = 4, 384, 384, 96
N = B * H * W

_i, _j = np.meshgrid(np.arange(H), np.arange(W), indexing="ij")
_src = (((41 * _i - 29 * _j) % H) * W + (-58 * _i + 41 * _j) % W).ravel()
_PERM = (_src[None, :] + (np.arange(B) * H * W)[:, None]).ravel().astype(np.int32)

_SC_INFO = plsc.get_sparse_core_info()
NC, NS = _SC_INFO.num_cores, _SC_INFO.num_subcores
NW = NC * NS
RPW = N // NW
K = 512
NCHUNK = RPW // K


def _sc_gather(table, perm):
    mesh = plsc.VectorSubcoreMesh(core_axis_name="c", subcore_axis_name="s")

    @functools.partial(
        pl.kernel,
        mesh=mesh,
        out_type=jax.ShapeDtypeStruct((N, C), jnp.float32),
        compiler_params=pltpu.CompilerParams(use_tc_tiling_on_sc=False),
        scratch_types=[
            pltpu.VMEM((K,), jnp.int32),
            pltpu.VMEM((K, C), jnp.float32),
            pltpu.SemaphoreType.DMA,
        ],
    )
    def k(table_hbm, perm_hbm, out_hbm, idx_v, rows_v, sem):
        wid = lax.axis_index("s") * NC + lax.axis_index("c")
        base = wid * RPW

        def chunk(n, carry):
            off = base + n * K
            pltpu.sync_copy(perm_hbm.at[pl.ds(off, K)], idx_v)
            pltpu.async_copy(table_hbm.at[idx_v], rows_v, sem).wait()
            pltpu.sync_copy(rows_v, out_hbm.at[pl.ds(off, K)])
            return carry

        lax.fori_loop(0, NCHUNK, chunk, 0)

    return k(table, perm)


def kernel(inputs):
    table = inputs.reshape(N, C)
    perm = jnp.asarray(_PERM)
    out = _sc_gather(table, perm)
    return out.reshape(B, H, W, C)


# speedup vs baseline: 1.3481x; 1.0336x over previous
import functools

import numpy as np
import jax
import jax.numpy as jnp
from jax import lax
from jax.experimental import pallas as pl
from jax.experimental.pallas import tpu as pltpu
from jax.experimental.pallas import tpu_sc as plsc

B, H, W, C = 4, 384, 384, 96
N = B * H * W

_i, _j = np.meshgrid(np.arange(H), np.arange(W), indexing="ij")
_src = (((41 * _i - 29 * _j) % H) * W + (-58 * _i + 41 * _j) % W).ravel()
_PERM = (_src[None, :] + (np.arange(B) * H * W)[:, None]).ravel().astype(np.int32)

_SC_INFO = plsc.get_sparse_core_info()
NC, NS = _SC_INFO.num_cores, _SC_INFO.num_subcores
NW = NC * NS
RPW = N // NW
K = 128
NCHUNK = RPW // K
NBUF = 8
LOOKAHEAD = 4


def _sc_gather(table, perm):
    mesh = plsc.VectorSubcoreMesh(core_axis_name="c", subcore_axis_name="s")

    @functools.partial(
        pl.kernel,
        mesh=mesh,
        out_type=jax.ShapeDtypeStruct((N, C), jnp.float32),
        compiler_params=pltpu.CompilerParams(use_tc_tiling_on_sc=False),
        scratch_types=(
            [pltpu.VMEM((NCHUNK, K), jnp.int32)]
            + [pltpu.VMEM((K, C), jnp.float32) for _ in range(NBUF)]
            + [pltpu.SemaphoreType.DMA for _ in range(2 * NBUF)]
        ),
    )
    def k(table_hbm, perm_hbm, out_hbm, idx_all, *rest):
        rows = rest[:NBUF]
        gsem = rest[NBUF:2 * NBUF]
        wsem = rest[2 * NBUF:]
        wid = lax.axis_index("s") * NC + lax.axis_index("c")
        base = wid * RPW

        pltpu.sync_copy(perm_hbm.at[wid], idx_all)

        def g_desc(n, s):
            return pltpu.make_async_copy(
                table_hbm.at[idx_all.at[n]], rows[s], gsem[s])

        def w_desc(n, s):
            return pltpu.make_async_copy(
                rows[s], out_hbm.at[pl.ds(base + n * K, K)], wsem[s])

        for s in range(LOOKAHEAD):
            g_desc(s, s).start()

        @pl.loop(0, NCHUNK, step=NBUF)
        def grp(n0):
            for s in range(NBUF):
                n = n0 + s
                g_desc(n, s).wait()
                w_desc(n, s).start()
                m = n + LOOKAHEAD
                sm = (s + LOOKAHEAD) % NBUF

                @pl.when(m < NCHUNK)
                def _issue():
                    @pl.when(m >= NBUF)
                    def _drain():
                        w_desc(m - NBUF, sm).wait()
                    g_desc(m, sm).start()

        for s in range(NBUF):
            w_desc(NCHUNK - NBUF + s, s).wait()

    return k(table, perm)


def kernel(inputs):
    table = inputs.reshape(N, C)
    perm = jnp.asarray(_PERM.reshape(NW, NCHUNK, K))
    out = _sc_gather(table, perm)
    return out.reshape(B, H, W, C)


# speedup vs baseline: 1.9075x; 1.4150x over previous
import functools

import numpy as np
import jax
import jax.numpy as jnp
from jax import lax
from jax.experimental import pallas as pl
from jax.experimental.pallas import tpu as pltpu
from jax.experimental.pallas import tpu_sc as plsc

B, H, W, C = 4, 384, 384, 96
N = B * H * W

_i, _j = np.meshgrid(np.arange(H), np.arange(W), indexing="ij")
_src = (((41 * _i - 29 * _j) % H) * W + (-58 * _i + 41 * _j) % W).ravel()
_PERM = (_src[None, :] + (np.arange(B) * H * W)[:, None]).ravel().astype(np.int32)

_SC_INFO = plsc.get_sparse_core_info()
NC, NS = _SC_INFO.num_cores, _SC_INFO.num_subcores
NW = NC * NS
RPW = N // NW
K = 128
NCHUNK = RPW // K
NBUF = 6
LOOKAHEAD = 3
CP = 128


def _sc_gather(table, perm):
    mesh = plsc.VectorSubcoreMesh(core_axis_name="c", subcore_axis_name="s")

    @functools.partial(
        pl.kernel,
        mesh=mesh,
        out_type=jax.ShapeDtypeStruct((N, CP), jnp.float32),
        compiler_params=pltpu.CompilerParams(use_tc_tiling_on_sc=False),
        scratch_types=(
            [pltpu.VMEM((NCHUNK, K), jnp.int32)]
            + [pltpu.VMEM((K, CP), jnp.float32) for _ in range(NBUF)]
            + [pltpu.SemaphoreType.DMA for _ in range(2 * NBUF)]
        ),
    )
    def k(table_hbm, perm_hbm, out_hbm, idx_all, *rest):
        rows = rest[:NBUF]
        gsem = rest[NBUF:2 * NBUF]
        wsem = rest[2 * NBUF:]
        wid = lax.axis_index("s") * NC + lax.axis_index("c")
        base = wid * RPW

        pltpu.sync_copy(perm_hbm.at[wid], idx_all)

        def g_desc(n, s):
            return pltpu.make_async_copy(
                table_hbm.at[idx_all.at[n]], rows[s], gsem[s])

        def w_desc(n, s):
            return pltpu.make_async_copy(
                rows[s], out_hbm.at[pl.ds(base + n * K, K)], wsem[s])

        for s in range(LOOKAHEAD):
            g_desc(s, s).start()

        @pl.loop(0, NCHUNK, step=NBUF)
        def grp(n0):
            for s in range(NBUF):
                n = n0 + s
                g_desc(n, s).wait()
                w_desc(n, s).start()
                m = n + LOOKAHEAD
                sm = (s + LOOKAHEAD) % NBUF

                @pl.when(m < NCHUNK)
                def _issue():
                    @pl.when(m >= NBUF)
                    def _drain():
                        w_desc(m - NBUF, sm).wait()
                    g_desc(m, sm).start()

        for s in range(NBUF):
            w_desc(NCHUNK - NBUF + s, s).wait()

    return k(table, perm)


def kernel(inputs):
    padded = jnp.pad(inputs, ((0, 0), (0, 0), (0, 0), (0, CP - C)))
    table = padded.reshape(N, CP)
    perm = jnp.asarray(_PERM.reshape(NW, NCHUNK, K))
    out = _sc_gather(table, perm)
    return out[:, :C].reshape(B, H, W, C)


# speedup vs baseline: 1.9126x; 1.0026x over previous
import functools

import numpy as np
import jax
import jax.numpy as jnp
from jax import lax
from jax.experimental import pallas as pl
from jax.experimental.pallas import tpu as pltpu
from jax.experimental.pallas import tpu_sc as plsc

B, H, W, C = 4, 384, 384, 96
N = B * H * W

_i, _j = np.meshgrid(np.arange(H), np.arange(W), indexing="ij")
_src = (((41 * _i - 29 * _j) % H) * W + (-58 * _i + 41 * _j) % W).ravel()
_PERM = (_src[None, :] + (np.arange(B) * H * W)[:, None]).ravel().astype(np.int32)

_SC_INFO = plsc.get_sparse_core_info()
NC, NS = _SC_INFO.num_cores, _SC_INFO.num_subcores
NW = NC * NS
RPW = N // NW
K = 128
NCHUNK = RPW // K
NBUF = 3
LOOKAHEAD = 2
CP = 128
NQ = C // 16


def _sc_gather(table, perm):
    mesh = plsc.VectorSubcoreMesh(core_axis_name="c", subcore_axis_name="s")

    @functools.partial(
        pl.kernel,
        mesh=mesh,
        out_type=jax.ShapeDtypeStruct((N, C), jnp.float32),
        compiler_params=pltpu.CompilerParams(use_tc_tiling_on_sc=True),
        scratch_types=(
            [pltpu.VMEM((NCHUNK, K), jnp.int32)]
            + [pltpu.VMEM((K, CP), jnp.float32) for _ in range(NBUF)]
            + [pltpu.VMEM((K, C), jnp.float32) for _ in range(NBUF)]
            + [pltpu.SemaphoreType.DMA for _ in range(2 * NBUF)]
        ),
    )
    def k(table_hbm, perm_hbm, out_hbm, idx_all, *rest):
        rows = rest[:NBUF]
        srcs = rest[NBUF:2 * NBUF]
        gsem = rest[2 * NBUF:3 * NBUF]
        wsem = rest[3 * NBUF:]
        wid = lax.axis_index("s") * NC + lax.axis_index("c")
        base = wid * RPW

        pltpu.sync_copy(perm_hbm.at[wid], idx_all)

        def g_desc(n, s):
            return pltpu.make_async_copy(
                table_hbm.at[idx_all.at[n]], rows[s], gsem[s])

        def w_desc(n, s):
            return pltpu.make_async_copy(
                srcs[s], out_hbm.at[pl.ds(base + n * K, K)], wsem[s])

        for s in range(LOOKAHEAD):
            g_desc(s, s).start()

        @pl.loop(0, NCHUNK, step=NBUF)
        def grp(n0):
            for s in range(NBUF):
                n = n0 + s
                g_desc(n, s).wait()

                @pl.when(n >= NBUF)
                def _drain():
                    w_desc(n - NBUF, s).wait()

                @pl.loop(0, K)
                def compact(p):
                    for q in range(NQ):
                        srcs[s][p, pl.ds(16 * q, 16)] = (
                            rows[s][p, pl.ds(16 * q, 16)])

                w_desc(n, s).start()
                m = n + LOOKAHEAD
                sm = (s + LOOKAHEAD) % NBUF

                @pl.when(m < NCHUNK)
                def _issue():
                    g_desc(m, sm).start()

        for s in range(NBUF):
            w_desc(NCHUNK - NBUF + s, s).wait()

    return k(table, perm)


def kernel(inputs):
    padded = jnp.pad(inputs, ((0, 0), (0, 0), (0, 0), (0, CP - C)))
    table = padded.reshape(N, CP)
    perm = jnp.asarray(_PERM.reshape(NW, NCHUNK, K))
    out = _sc_gather(table, perm)
    return out.reshape(B, H, W, C)
